# Initial kernel scaffold; baseline (speedup 1.0000x reference)
import jax
import jax.numpy as jnp
from jax.experimental import pallas as pl


def kernel(h_real, h_imag, sel_W1, sel_b1, sel_W2, sel_b2, net_W1, net_b1, net_W2, net_b2, net_W3, net_b3):
    raise NotImplementedError("write your pallas kernel here")


# speedup vs baseline: 6.7806x; 6.7806x over previous
import functools

import jax
import jax.numpy as jnp
from jax.experimental import pallas as pl
from jax.experimental.pallas import tpu as pltpu

B = 128
K = 8192
M = 1024
H0 = 4096
H1 = 4096


def _mm_body(x_ref, w_ref, b_ref, extra_ref, out_ref, *, nk, relu):
    k = pl.program_id(1)

    @pl.when(k == 0)
    def _():
        out_ref[...] = jnp.zeros_like(out_ref)

    out_ref[...] += jnp.dot(x_ref[...], w_ref[...],
                            preferred_element_type=jnp.float32)

    @pl.when(k == nk - 1)
    def _():
        acc = out_ref[...] + b_ref[...]
        if extra_ref is not None:
            acc = acc + extra_ref[...]
        if relu:
            acc = jnp.maximum(acc, 0.0)
        out_ref[...] = acc


def _mm(x, W, b, *, relu, extra=None, bn=1024, bk=2048):
    Bx, Kin = x.shape
    N = W.shape[1]
    bk = min(bk, Kin)
    bn = min(bn, N)
    nk = Kin // bk
    grid = (N // bn, nk)
    b2 = b.reshape(1, N)

    in_specs = [
        pl.BlockSpec((Bx, bk), lambda n, k: (0, k)),
        pl.BlockSpec((bk, bn), lambda n, k: (k, n)),
        pl.BlockSpec((1, bn), lambda n, k: (0, n)),
    ]
    args = [x, W, b2]
    if extra is not None:
        in_specs.append(pl.BlockSpec((Bx, bn), lambda n, k: (0, n)))
        args.append(extra)
        body = functools.partial(_mm_body, nk=nk, relu=relu)
    else:
        def body(x_ref, w_ref, b_ref, out_ref):
            _mm_body(x_ref, w_ref, b_ref, None, out_ref, nk=nk, relu=relu)

    return pl.pallas_call(
        body,
        grid=grid,
        in_specs=in_specs,
        out_specs=pl.BlockSpec((Bx, bn), lambda n, k: (0, n)),
        out_shape=jax.ShapeDtypeStruct((Bx, N), jnp.float32),
        compiler_params=pltpu.CompilerParams(
            dimension_semantics=("parallel", "arbitrary")),
    )(*args)


def _gumbel_noise():
    u = jax.random.uniform(jax.random.key(42), (B, K), dtype=jnp.float32)
    return -jnp.log(-jnp.log(u + 1e-10) + 1e-10)


def kernel(h_real, h_imag, sel_W1, sel_b1, sel_W2, sel_b2,
           net_W1, net_b1, net_W2, net_b2, net_W3, net_b3):
    h_cat = jnp.concatenate([h_real, h_imag], axis=-1)
    hid = jax.nn.relu(h_cat @ sel_W1 + sel_b1)
    g = (hid @ sel_W2 + sel_b2) + _gumbel_noise()

    _, idx = jax.lax.top_k(g, M)
    h_r = jnp.take_along_axis(h_real, idx, axis=-1)
    h_i = jnp.take_along_axis(h_imag, idx, axis=-1)

    h_sel = jnp.concatenate([h_r, h_i], axis=-1)
    x = _mm(h_sel, net_W1, net_b1, relu=True)
    x = _mm(x, net_W2, net_b2, relu=True)
    out = _mm(x, net_W3, net_b3, relu=False)
    return out, idx


# speedup vs baseline: 6.8349x; 1.0080x over previous
import functools

import jax
import jax.numpy as jnp
from jax.experimental import pallas as pl
from jax.experimental.pallas import tpu as pltpu

B = 128
K = 8192
M = 1024
H0 = 4096
H1 = 4096


def _mm_body(x_ref, w_ref, b_ref, extra_ref, out_ref, *, nk, relu):
    k = pl.program_id(1)

    @pl.when(k == 0)
    def _():
        out_ref[...] = jnp.zeros_like(out_ref)

    out_ref[...] += jnp.dot(x_ref[...], w_ref[...],
                            preferred_element_type=jnp.float32)

    @pl.when(k == nk - 1)
    def _():
        acc = out_ref[...] + b_ref[...]
        if extra_ref is not None:
            acc = acc + extra_ref[...]
        if relu:
            acc = jnp.maximum(acc, 0.0)
        out_ref[...] = acc


def _mm(x, W, b, *, relu, extra=None, bn=2048, bk=2048):
    Bx, Kin = x.shape
    N = W.shape[1]
    bk = min(bk, Kin)
    bn = min(bn, N)
    nk = Kin // bk
    grid = (N // bn, nk)
    b2 = b.reshape(1, N)

    in_specs = [
        pl.BlockSpec((Bx, bk), lambda n, k: (0, k)),
        pl.BlockSpec((bk, bn), lambda n, k: (k, n)),
        pl.BlockSpec((1, bn), lambda n, k: (0, n)),
    ]
    args = [x, W, b2]
    if extra is not None:
        in_specs.append(pl.BlockSpec((Bx, bn), lambda n, k: (0, n)))
        args.append(extra)
        body = functools.partial(_mm_body, nk=nk, relu=relu)
    else:
        def body(x_ref, w_ref, b_ref, out_ref):
            _mm_body(x_ref, w_ref, b_ref, None, out_ref, nk=nk, relu=relu)

    return pl.pallas_call(
        body,
        grid=grid,
        in_specs=in_specs,
        out_specs=pl.BlockSpec((Bx, bn), lambda n, k: (0, n)),
        out_shape=jax.ShapeDtypeStruct((Bx, N), jnp.float32),
        compiler_params=pltpu.CompilerParams(
            dimension_semantics=("parallel", "arbitrary")),
    )(*args)


def _gumbel_noise():
    u = jax.random.uniform(jax.random.key(42), (B, K), dtype=jnp.float32)
    return -jnp.log(-jnp.log(u + 1e-10) + 1e-10)


def kernel(h_real, h_imag, sel_W1, sel_b1, sel_W2, sel_b2,
           net_W1, net_b1, net_W2, net_b2, net_W3, net_b3):
    h_cat = jnp.concatenate([h_real, h_imag], axis=-1)
    hid = jax.nn.relu(h_cat @ sel_W1 + sel_b1)
    g = (hid @ sel_W2 + sel_b2) + _gumbel_noise()

    _, idx = jax.lax.top_k(g, M)
    h_r = jnp.take_along_axis(h_real, idx, axis=-1)
    h_i = jnp.take_along_axis(h_imag, idx, axis=-1)

    h_sel = jnp.concatenate([h_r, h_i], axis=-1)
    x = _mm(h_sel, net_W1, net_b1, relu=True)
    x = _mm(x, net_W2, net_b2, relu=True)
    out = _mm(x, net_W3, net_b3, relu=False)
    return out, idx
